# baseline (speedup 1.0000x reference)
import functools
import math

import jax
import jax.numpy as jnp
from jax import lax
from jax.experimental import pallas as pl
from jax.experimental.pallas import tpu as pltpu
from jax.experimental.pallas import tpu_sc as plsc

D_MODEL = 64
LANES = 16
NUM_CORES = 2
NUM_SUBCORES = 16
NUM_WORKERS = NUM_CORES * NUM_SUBCORES
SCALE = math.sqrt(D_MODEL)

TS = 2
ROWS = TS * 128


@functools.lru_cache(maxsize=None)
def _build(n_t: int, n_b: int, vocab: int):
    assert n_t % 8 == 0 and n_b % 128 == 0
    tr_n, bc_n = n_t // 8, n_b // 128
    h_n = 8 // TS
    n_tasks = tr_n * bc_n * h_n
    assert n_tasks % NUM_WORKERS == 0
    per_worker = n_tasks // NUM_WORKERS

    mesh = plsc.VectorSubcoreMesh(
        core_axis_name="c", subcore_axis_name="s",
        num_cores=NUM_CORES, num_subcores=NUM_SUBCORES)

    @functools.partial(
        pl.kernel,
        out_type=jax.ShapeDtypeStruct((n_t, 8, bc_n, 8, 128), jnp.float32),
        mesh=mesh,
        scratch_types=[
            pltpu.VMEM((ROWS,), jnp.int32),
            pltpu.VMEM((ROWS, D_MODEL), jnp.float32),
            pltpu.VMEM((TS * 8, 8, 128), jnp.float32),
            pltpu.SemaphoreType.DMA,
            pltpu.SemaphoreType.DMA,
        ],
        compiler_params=pltpu.CompilerParams(
            use_tc_tiling_on_sc=False, needs_layout_passes=False),
    )
    def emb_kernel(table_hbm, xv_hbm, out_hbm, idx_v, rows_v, out_v, gsem, osem):
        wid = lax.axis_index("s") * NUM_CORES + lax.axis_index("c")

        def task_body(ti, carry):
            iota = lax.iota(jnp.int32, LANES)
            g = wid * per_worker + ti
            tr = g // (bc_n * h_n)
            rem = g % (bc_n * h_n)
            bc = rem // h_n
            h = rem % h_n
            base = ((tr * bc_n + bc) * 8 + h * TS) * 128
            pltpu.sync_copy(xv_hbm.at[pl.ds(base, ROWS)], idx_v)
            pltpu.async_copy(table_hbm.at[idx_v], rows_v, gsem).wait()

            for tl in range(TS):
                for bg in range(8):
                    row_ids = tl * 128 + bg * 16 + iota
                    for dr in range(8):
                        for ds in range(8):
                            col_ids = jnp.full((LANES,), dr * 8 + ds, jnp.int32)
                            v = plsc.load_gather(rows_v, [row_ids, col_ids])
                            out_v[tl * 8 + dr, ds, pl.ds(bg * 16, LANES)] = (
                                v * SCALE)

            t0 = tr * 8 + h * TS
            for tl in range(TS):
                for dr in range(8):
                    cp = pltpu.make_async_copy(
                        out_v.at[tl * 8 + dr],
                        out_hbm.at[t0 + tl, dr, bc],
                        osem)
                    cp.start()
                    cp.wait()
            return carry

        lax.fori_loop(0, per_worker, task_body, 0)

    return emb_kernel


def kernel(x, table):
    n_b, n_t = x.shape
    vocab = table.shape[0]
    xv = (x.transpose(1, 0)
           .reshape(n_t // 8, 8, n_b // 128, 128)
           .transpose(0, 2, 1, 3)
           .reshape(n_t * n_b))
    out5 = _build(n_t, n_b, vocab)(table, xv)
    out = (out5.transpose(2, 4, 0, 1, 3)
               .reshape(n_b, n_t, D_MODEL))
    return out


# speedup vs baseline: 1.1374x; 1.1374x over previous
import functools
import math

import jax
import jax.numpy as jnp
from jax import lax
from jax.experimental import pallas as pl
from jax.experimental.pallas import tpu as pltpu
from jax.experimental.pallas import tpu_sc as plsc

D_MODEL = 64
LANES = 16
NUM_CORES = 2
NUM_SUBCORES = 16
NUM_WORKERS = NUM_CORES * NUM_SUBCORES
SCALE = math.sqrt(D_MODEL)

BCC = 4
ROWS = BCC * 128


@functools.lru_cache(maxsize=None)
def _build(n_t: int, n_b: int):
    assert n_t % 8 == 0 and n_b % (BCC * 128) == 0
    tr_n, bc_n = n_t // 8, n_b // 128
    ch_n = bc_n // BCC
    n_tasks = n_t * ch_n
    assert n_tasks % NUM_WORKERS == 0
    per_worker = n_tasks // NUM_WORKERS

    mesh = plsc.VectorSubcoreMesh(
        core_axis_name="c", subcore_axis_name="s",
        num_cores=NUM_CORES, num_subcores=NUM_SUBCORES)

    @functools.partial(
        pl.kernel,
        out_type=jax.ShapeDtypeStruct((n_t, 8, bc_n, 8, 128), jnp.float32),
        mesh=mesh,
        scratch_types=[
            pltpu.VMEM((2, ROWS), jnp.int32),
            pltpu.VMEM((2, ROWS, D_MODEL), jnp.float32),
            pltpu.VMEM((8, BCC, 8, 128), jnp.float32),
            pltpu.SemaphoreType.DMA,
            pltpu.SemaphoreType.DMA,
            pltpu.SemaphoreType.DMA,
            pltpu.SemaphoreType.DMA,
            pltpu.SemaphoreType.DMA,
        ],
        compiler_params=pltpu.CompilerParams(
            use_tc_tiling_on_sc=False, needs_layout_passes=False),
    )
    def emb_kernel(table_hbm, xv_hbm, out_hbm, idx_v, rows_v, out_v,
                   isem0, isem1, gsem0, gsem1, osem):
        wid = lax.axis_index("s") * NUM_CORES + lax.axis_index("c")
        g0 = wid * per_worker

        def coords(g):
            t = g // ch_n
            c0 = (g % ch_n) * BCC
            return t, c0

        def idx_src_base(t, c0, cc):
            tr, ts = t // 8, t % 8
            return ((tr * bc_n + c0 + cc) * 8 + ts) * 128

        def fire_idx(g, buf, sem):
            t, c0 = coords(g)
            for cc in range(BCC):
                pltpu.async_copy(
                    xv_hbm.at[pl.ds(idx_src_base(t, c0, cc), 128)],
                    idx_v.at[buf, pl.ds(cc * 128, 128)], sem)

        def wait_idx(g, buf, sem):
            t, c0 = coords(g)
            for cc in range(BCC):
                pltpu.make_async_copy(
                    xv_hbm.at[pl.ds(idx_src_base(t, c0, cc), 128)],
                    idx_v.at[buf, pl.ds(cc * 128, 128)], sem).wait()

        def out_tile_copy(g, dr, sem):
            t, c0 = coords(g)
            return pltpu.make_async_copy(
                out_v.at[dr], out_hbm.at[t, dr, pl.ds(c0, BCC)], sem)

        isems = (isem0, isem1)
        gsems = (gsem0, gsem1)

        fire_idx(g0, 0, isems[0])
        fire_idx(g0 + 1, 1, isems[1])

        def phase(ti, p):
            @pl.when(ti < per_worker)
            def _gather():
                wait_idx(g0 + ti, p, isems[p])
                pltpu.async_copy(
                    table_hbm.at[idx_v.at[p]], rows_v.at[p], gsems[p])

            @pl.when((ti >= 1) & (ti <= per_worker))
            def _process():
                q = 1 - p
                g = g0 + ti - 1
                pltpu.make_async_copy(
                    table_hbm.at[idx_v.at[q]], rows_v.at[q], gsems[q]).wait()

                @pl.when(ti + 1 < per_worker)
                def _prefetch():
                    fire_idx(g0 + ti + 1, q, isems[q])

                @pl.when(ti >= 2)
                def _drain():
                    for dr in range(8):
                        out_tile_copy(g, dr, osem).wait()

                def tr_body(u, c2):
                    iota = lax.iota(jnp.int32, LANES)
                    cc = u // 8
                    bg = u % 8
                    row_ids = cc * 128 + bg * 16 + iota
                    for dr in range(8):
                        for ds in range(8):
                            col_ids = jnp.full((LANES,), dr * 8 + ds,
                                               jnp.int32)
                            v = plsc.load_gather(
                                rows_v.at[q], [row_ids, col_ids])
                            out_v[dr, cc, ds, pl.ds(bg * 16, LANES)] = (
                                v * SCALE)
                    return c2

                lax.fori_loop(0, BCC * 8, tr_body, 0)

                for dr in range(8):
                    out_tile_copy(g, dr, osem).start()

        def step(ti2, carry):
            phase(ti2 * 2, 0)
            phase(ti2 * 2 + 1, 1)
            return carry

        lax.fori_loop(0, per_worker // 2 + 1, step, 0)

        for dr in range(8):
            out_tile_copy(g0 + per_worker - 1, dr, osem).wait()

    return emb_kernel


def kernel(x, table):
    n_b, n_t = x.shape
    xv = (x.transpose(1, 0)
           .reshape(n_t // 8, 8, n_b // 128, 128)
           .transpose(0, 2, 1, 3)
           .reshape(n_t * n_b))
    out5 = _build(n_t, n_b)(table, xv)
    return out5.transpose(2, 4, 0, 1, 3).reshape(n_b, n_t, D_MODEL)


# speedup vs baseline: 1.5617x; 1.3731x over previous
import functools
import math

import jax
import jax.numpy as jnp
from jax import lax
from jax.experimental import pallas as pl
from jax.experimental.pallas import tpu as pltpu
from jax.experimental.pallas import tpu_sc as plsc

D_MODEL = 64
LANES = 16
NUM_CORES = 2
NUM_SUBCORES = 16
NUM_WORKERS = NUM_CORES * NUM_SUBCORES
SCALE = math.sqrt(D_MODEL)

BCC = 4
ROWS = BCC * 128


@functools.lru_cache(maxsize=None)
def _build(n_t: int, n_b: int):
    assert n_t % 8 == 0 and n_b % (BCC * 128) == 0
    tr_n, bc_n = n_t // 8, n_b // 128
    ch_n = bc_n // BCC
    n_tasks = n_t * ch_n
    assert n_tasks % NUM_WORKERS == 0
    per_worker = n_tasks // NUM_WORKERS

    mesh = plsc.VectorSubcoreMesh(
        core_axis_name="c", subcore_axis_name="s",
        num_cores=NUM_CORES, num_subcores=NUM_SUBCORES)

    @functools.partial(
        pl.kernel,
        out_type=jax.ShapeDtypeStruct((n_t, 8, bc_n, 8, 128), jnp.float32),
        mesh=mesh,
        scratch_types=[
            pltpu.VMEM((2, ROWS), jnp.int32),
            pltpu.VMEM((2, ROWS, D_MODEL), jnp.float32),
            pltpu.VMEM((8, BCC, 8, 128), jnp.float32),
            pltpu.SemaphoreType.DMA,
            pltpu.SemaphoreType.DMA,
            pltpu.SemaphoreType.DMA,
            pltpu.SemaphoreType.DMA,
            pltpu.SemaphoreType.DMA,
        ],
        compiler_params=pltpu.CompilerParams(
            use_tc_tiling_on_sc=False, needs_layout_passes=False),
    )
    def emb_kernel(table_hbm, xv_hbm, out_hbm, idx_v, rows_v, out_v,
                   isem0, isem1, gsem0, gsem1, osem):
        wid = lax.axis_index("s") * NUM_CORES + lax.axis_index("c")
        g0 = wid * per_worker

        def coords(g):
            t = g // ch_n
            c0 = (g % ch_n) * BCC
            return t, c0

        def idx_src_base(t, c0, cc):
            tr, ts = t // 8, t % 8
            return ((tr * bc_n + c0 + cc) * 8 + ts) * 128

        def fire_idx(g, buf, sem):
            t, c0 = coords(g)
            for cc in range(BCC):
                pltpu.async_copy(
                    xv_hbm.at[pl.ds(idx_src_base(t, c0, cc), 128)],
                    idx_v.at[buf, pl.ds(cc * 128, 128)], sem)

        def wait_idx(g, buf, sem):
            t, c0 = coords(g)
            for cc in range(BCC):
                pltpu.make_async_copy(
                    xv_hbm.at[pl.ds(idx_src_base(t, c0, cc), 128)],
                    idx_v.at[buf, pl.ds(cc * 128, 128)], sem).wait()

        def out_tile_copy(g, dr, sem):
            t, c0 = coords(g)
            return pltpu.make_async_copy(
                out_v.at[dr], out_hbm.at[t, dr, pl.ds(c0, BCC)], sem)

        isems = (isem0, isem1)
        gsems = (gsem0, gsem1)

        fire_idx(g0, 0, isems[0])
        fire_idx(g0 + 1, 1, isems[1])

        def phase(ti, p):
            @pl.when(ti < per_worker)
            def _gather():
                wait_idx(g0 + ti, p, isems[p])
                pltpu.async_copy(
                    table_hbm.at[idx_v.at[p]], rows_v.at[p], gsems[p])

            @pl.when((ti >= 1) & (ti <= per_worker))
            def _process():
                q = 1 - p
                g = g0 + ti - 1
                pltpu.make_async_copy(
                    table_hbm.at[idx_v.at[q]], rows_v.at[q], gsems[q]).wait()

                @pl.when(ti + 1 < per_worker)
                def _prefetch():
                    fire_idx(g0 + ti + 1, q, isems[q])

                @pl.when(ti >= 2)
                def _drain():
                    for dr in range(8):
                        out_tile_copy(g, dr, osem).wait()

                @plsc.parallel_loop(0, BCC * 8, 1, unroll=2)
                def tr_body(u):
                    iota = lax.iota(jnp.int32, LANES)
                    cc = u // 8
                    bg = u % 8
                    row_ids = cc * 128 + bg * 16 + iota
                    for dr in range(8):
                        for ds in range(8):
                            col_ids = jnp.full((LANES,), dr * 8 + ds,
                                               jnp.int32)
                            v = plsc.load_gather(
                                rows_v.at[q], [row_ids, col_ids])
                            out_v[dr, cc, ds, pl.ds(bg * 16, LANES)] = (
                                v * SCALE)

                for dr in range(8):
                    out_tile_copy(g, dr, osem).start()

        def step(ti2, carry):
            phase(ti2 * 2, 0)
            phase(ti2 * 2 + 1, 1)
            return carry

        lax.fori_loop(0, per_worker // 2 + 1, step, 0)

        for dr in range(8):
            out_tile_copy(g0 + per_worker - 1, dr, osem).wait()

    return emb_kernel


def kernel(x, table):
    n_b, n_t = x.shape
    xv = (x.transpose(1, 0)
           .reshape(n_t // 8, 8, n_b // 128, 128)
           .transpose(0, 2, 1, 3)
           .reshape(n_t * n_b))
    out5 = _build(n_t, n_b)(table, xv)
    return out5.transpose(2, 4, 0, 1, 3).reshape(n_b, n_t, D_MODEL)


# speedup vs baseline: 1.7650x; 1.1302x over previous
import functools
import math

import jax
import jax.numpy as jnp
from jax import lax
from jax.experimental import pallas as pl
from jax.experimental.pallas import tpu as pltpu
from jax.experimental.pallas import tpu_sc as plsc

D_MODEL = 64
LANES = 16
NUM_CORES = 2
NUM_SUBCORES = 16
NUM_WORKERS = NUM_CORES * NUM_SUBCORES
SCALE = math.sqrt(D_MODEL)

BCC = 4
ROWS = BCC * 128
NPAD = 4


@functools.lru_cache(maxsize=None)
def _build(n_t: int, n_b: int):
    assert n_t % 8 == 0 and n_b % (BCC * 128) == 0
    tr_n, bc_n = n_t // 8, n_b // 128
    ch_n = bc_n // BCC
    n_tasks = n_t * ch_n
    assert n_tasks % NUM_WORKERS == 0
    per_worker = n_tasks // NUM_WORKERS

    mesh = plsc.VectorSubcoreMesh(
        core_axis_name="c", subcore_axis_name="s",
        num_cores=NUM_CORES, num_subcores=NUM_SUBCORES)

    @functools.partial(
        pl.kernel,
        out_type=jax.ShapeDtypeStruct((n_t, 8, bc_n, 8, 128), jnp.float32),
        mesh=mesh,
        scratch_types=[
            pltpu.VMEM((2, ROWS), jnp.int32),
            pltpu.VMEM((2, ROWS, D_MODEL), jnp.float32),
            pltpu.VMEM((NPAD * LANES, 65), jnp.float32),
            pltpu.VMEM((8, BCC, 8, 128), jnp.float32),
            pltpu.SemaphoreType.DMA,
            pltpu.SemaphoreType.DMA,
            pltpu.SemaphoreType.DMA,
            pltpu.SemaphoreType.DMA,
            pltpu.SemaphoreType.DMA,
        ],
        compiler_params=pltpu.CompilerParams(
            use_tc_tiling_on_sc=False, needs_layout_passes=False),
    )
    def emb_kernel(table_hbm, xv_hbm, out_hbm, idx_v, rows_v, pad_v, out_v,
                   isem0, isem1, gsem0, gsem1, osem):
        wid = lax.axis_index("s") * NUM_CORES + lax.axis_index("c")
        g0 = wid * per_worker

        def coords(g):
            t = g // ch_n
            c0 = (g % ch_n) * BCC
            return t, c0

        def idx_src_base(t, c0, cc):
            tr, ts = t // 8, t % 8
            return ((tr * bc_n + c0 + cc) * 8 + ts) * 128

        def fire_idx(g, buf, sem):
            t, c0 = coords(g)
            for cc in range(BCC):
                pltpu.async_copy(
                    xv_hbm.at[pl.ds(idx_src_base(t, c0, cc), 128)],
                    idx_v.at[buf, pl.ds(cc * 128, 128)], sem)

        def wait_idx(g, buf, sem):
            t, c0 = coords(g)
            for cc in range(BCC):
                pltpu.make_async_copy(
                    xv_hbm.at[pl.ds(idx_src_base(t, c0, cc), 128)],
                    idx_v.at[buf, pl.ds(cc * 128, 128)], sem).wait()

        def out_tile_copy(g, dr, sem):
            t, c0 = coords(g)
            return pltpu.make_async_copy(
                out_v.at[dr], out_hbm.at[t, dr, pl.ds(c0, BCC)], sem)

        isems = (isem0, isem1)
        gsems = (gsem0, gsem1)

        fire_idx(g0, 0, isems[0])
        fire_idx(g0 + 1, 1, isems[1])

        def phase(ti, p):
            @pl.when(ti < per_worker)
            def _gather():
                wait_idx(g0 + ti, p, isems[p])

                @plsc.parallel_loop(0, ROWS // LANES, 1, unroll=4)
                def _dbl(j):
                    sl = pl.ds(j * LANES, LANES)
                    idx_v[p, sl] = idx_v[p, sl] * 2

                pltpu.async_copy(
                    table_hbm.at[idx_v.at[p]], rows_v.at[p], gsems[p])

            @pl.when((ti >= 1) & (ti <= per_worker))
            def _process():
                q = 1 - p
                g = g0 + ti - 1
                pltpu.make_async_copy(
                    table_hbm.at[idx_v.at[q]], rows_v.at[q], gsems[q]).wait()

                @pl.when(ti + 1 < per_worker)
                def _prefetch():
                    fire_idx(g0 + ti + 1, q, isems[q])

                @pl.when(ti >= 2)
                def _drain():
                    for dr in range(8):
                        out_tile_copy(g, dr, osem).wait()

                @plsc.parallel_loop(0, BCC * 8, 1, unroll=2)
                def tr_body(u):
                    iota = lax.iota(jnp.int32, LANES)
                    cc = u // 8
                    bg = u % 8
                    r0 = u * LANES
                    m0 = (u % NPAD) * LANES
                    for r in range(LANES):
                        for g4 in range(4):
                            sl = pl.ds(g4 * LANES, LANES)
                            pad_v[m0 + r, sl] = rows_v[q, r0 + r, sl] * SCALE
                    row_ids = m0 + iota
                    for dr in range(8):
                        for ds in range(8):
                            col_ids = jnp.full((LANES,), dr * 8 + ds,
                                               jnp.int32)
                            v = plsc.load_gather(pad_v, [row_ids, col_ids])
                            out_v[dr, cc, ds, pl.ds(bg * 16, LANES)] = v

                for dr in range(8):
                    out_tile_copy(g, dr, osem).start()

        def step(ti2, carry):
            phase(ti2 * 2, 0)
            phase(ti2 * 2 + 1, 1)
            return carry

        lax.fori_loop(0, per_worker // 2 + 1, step, 0)

        for dr in range(8):
            out_tile_copy(g0 + per_worker - 1, dr, osem).wait()

    return emb_kernel


def kernel(x, table):
    n_b, n_t = x.shape
    vocab, d = table.shape
    tp = jnp.pad(table, ((0, 0), (0, 128 - d))).reshape(2 * vocab, d)
    xv = (x.transpose(1, 0)
           .reshape(n_t // 8, 8, n_b // 128, 128)
           .transpose(0, 2, 1, 3)
           .reshape(n_t * n_b))
    out5 = _build(n_t, n_b)(tp, xv)
    return out5.transpose(2, 4, 0, 1, 3).reshape(n_b, n_t, D_MODEL)


# speedup vs baseline: 2.2592x; 1.2800x over previous
import functools
import math

import jax
import jax.numpy as jnp
from jax import lax
from jax.experimental import pallas as pl
from jax.experimental.pallas import tpu as pltpu
from jax.experimental.pallas import tpu_sc as plsc

D_MODEL = 64
LANES = 16
NUM_CORES = 2
NUM_SUBCORES = 16
NUM_WORKERS = NUM_CORES * NUM_SUBCORES
SCALE = math.sqrt(D_MODEL)

BCC = 2
ROWS = BCC * 128


@functools.lru_cache(maxsize=None)
def _build(n_t: int, n_b: int):
    assert n_t % 8 == 0 and n_b % (BCC * 128) == 0
    tr_n, bc_n = n_t // 8, n_b // 128
    ch_n = bc_n // BCC
    n_tasks = n_t * ch_n
    assert n_tasks % NUM_WORKERS == 0
    per_worker = n_tasks // NUM_WORKERS

    mesh = plsc.VectorSubcoreMesh(
        core_axis_name="c", subcore_axis_name="s",
        num_cores=NUM_CORES, num_subcores=NUM_SUBCORES)

    @functools.partial(
        pl.kernel,
        out_type=jax.ShapeDtypeStruct((n_t, 8, bc_n, 8, 128), jnp.float32),
        mesh=mesh,
        scratch_types=[
            pltpu.VMEM((2, ROWS), jnp.int32),
            pltpu.VMEM((2, ROWS, D_MODEL), jnp.float32),
            pltpu.VMEM((ROWS, 65), jnp.float32),
            pltpu.VMEM((8, BCC, 8, 128), jnp.float32),
            pltpu.SemaphoreType.DMA,
            pltpu.SemaphoreType.DMA,
            pltpu.SemaphoreType.DMA,
            pltpu.SemaphoreType.DMA,
            pltpu.SemaphoreType.DMA,
        ],
        compiler_params=pltpu.CompilerParams(
            use_tc_tiling_on_sc=False, needs_layout_passes=False),
    )
    def emb_kernel(table_hbm, xv_hbm, out_hbm, idx_v, rows_v, pad_v, out_v,
                   isem0, isem1, gsem0, gsem1, osem):
        wid = lax.axis_index("s") * NUM_CORES + lax.axis_index("c")
        g0 = wid * per_worker

        def coords(g):
            t = g // ch_n
            c0 = (g % ch_n) * BCC
            return t, c0

        def idx_src_base(t, c0, cc):
            tr, ts = t // 8, t % 8
            return ((tr * bc_n + c0 + cc) * 8 + ts) * 128

        def fire_idx(g, buf, sem):
            t, c0 = coords(g)
            for cc in range(BCC):
                pltpu.async_copy(
                    xv_hbm.at[pl.ds(idx_src_base(t, c0, cc), 128)],
                    idx_v.at[buf, pl.ds(cc * 128, 128)], sem)

        def wait_idx(g, buf, sem):
            t, c0 = coords(g)
            for cc in range(BCC):
                pltpu.make_async_copy(
                    xv_hbm.at[pl.ds(idx_src_base(t, c0, cc), 128)],
                    idx_v.at[buf, pl.ds(cc * 128, 128)], sem).wait()

        def out_tile_copy(g, dr, sem):
            t, c0 = coords(g)
            return pltpu.make_async_copy(
                out_v.at[dr], out_hbm.at[t, dr, pl.ds(c0, BCC)], sem)

        isems = (isem0, isem1)
        gsems = (gsem0, gsem1)

        fire_idx(g0, 0, isems[0])
        fire_idx(g0 + 1, 1, isems[1])

        def phase(ti, p):
            @pl.when(ti < per_worker)
            def _gather():
                wait_idx(g0 + ti, p, isems[p])

                @plsc.parallel_loop(0, ROWS // LANES, 1, unroll=4)
                def _dbl(j):
                    sl = pl.ds(j * LANES, LANES)
                    idx_v[p, sl] = idx_v[p, sl] * 2

                pltpu.async_copy(
                    table_hbm.at[idx_v.at[p]], rows_v.at[p], gsems[p])

            @pl.when((ti >= 1) & (ti <= per_worker))
            def _process():
                q = 1 - p
                g = g0 + ti - 1
                pltpu.make_async_copy(
                    table_hbm.at[idx_v.at[q]], rows_v.at[q], gsems[q]).wait()

                @pl.when(ti + 1 < per_worker)
                def _prefetch():
                    fire_idx(g0 + ti + 1, q, isems[q])

                @pl.when(ti >= 2)
                def _drain():
                    for dr in range(8):
                        out_tile_copy(g, dr, osem).wait()

                @plsc.parallel_loop(0, BCC * 8, 1, unroll=1)
                def stage_body(u):
                    r0 = u * LANES
                    for r in range(LANES):
                        for g4 in range(4):
                            sl = pl.ds(g4 * LANES, LANES)
                            pad_v[r0 + r, sl] = rows_v[q, r0 + r, sl] * SCALE

                @plsc.parallel_loop(0, BCC * 8, 1, unroll=1)
                def tr_body(u):
                    iota = lax.iota(jnp.int32, LANES)
                    cc = u // 8
                    bg = u % 8
                    row_ids = u * LANES + iota
                    for dr in range(8):
                        for ds in range(8):
                            col_ids = jnp.full((LANES,), dr * 8 + ds,
                                               jnp.int32)
                            v = plsc.load_gather(pad_v, [row_ids, col_ids])
                            out_v[dr, cc, ds, pl.ds(bg * 16, LANES)] = v

                for dr in range(8):
                    out_tile_copy(g, dr, osem).start()

        def step(ti2, carry):
            phase(ti2 * 2, 0)
            phase(ti2 * 2 + 1, 1)
            return carry

        lax.fori_loop(0, per_worker // 2 + 1, step, 0)

        for dr in range(8):
            out_tile_copy(g0 + per_worker - 1, dr, osem).wait()

    return emb_kernel


def kernel(x, table):
    n_b, n_t = x.shape
    vocab, d = table.shape
    tp = jnp.pad(table, ((0, 0), (0, 128 - d))).reshape(2 * vocab, d)
    xv = (x.transpose(1, 0)
           .reshape(n_t // 8, 8, n_b // 128, 128)
           .transpose(0, 2, 1, 3)
           .reshape(n_t * n_b))
    out5 = _build(n_t, n_b)(tp, xv)
    return out5.transpose(2, 4, 0, 1, 3).reshape(n_b, n_t, D_MODEL)
